# Initial kernel scaffold; baseline (speedup 1.0000x reference)
import jax
import jax.numpy as jnp
from jax.experimental import pallas as pl


def kernel(x, edge_index, feature_scale, feature_bias, bn0_g, bn0_b, W1, b1, bn1_g, bn1_b, ln1_g, ln1_b, W2, b2, bn2_g, bn2_b, ln2_g, ln2_b, Wo, bo):
    raise NotImplementedError("write your pallas kernel here")


# speedup vs baseline: 8.0324x; 8.0324x over previous
import functools

import jax
import jax.numpy as jnp
import numpy as np
from jax import lax
from jax.experimental import pallas as pl
from jax.experimental.pallas import tpu as pltpu
from jax.experimental.pallas import tpu_sc as plsc

_N = 10000
_E = 320000
_D_IN = 128
_D_H = 256
_EPS = 1e-5

_NC = 2
_NS = 16
_NPAD = 10240
_RPT = _NPAD // _NS
_CH = 128

_EPT = _E // _NS
_SS = 16
_NCH_A = 160
_NSUP = _NCH_A // _SS
_EPT_PAD = _NCH_A * _CH

_EPW = _E // (_NC * _NS)
_NCH_D = 80
_NSUP_D = _NCH_D // _SS
_EPW_PAD = _NCH_D * _CH

_R = 1000
_GRID = _N // _R

_sc_mesh = plsc.VectorSubcoreMesh(core_axis_name="c", subcore_axis_name="s")


@functools.partial(
    pl.kernel,
    out_type=jax.ShapeDtypeStruct((_NC, _NPAD, 128), jnp.float32),
    mesh=_sc_mesh,
    scratch_types=[
        pltpu.VMEM((_SS, _CH), jnp.int32),
        pltpu.VMEM((_CH, 128), jnp.float32),
        pltpu.VMEM_SHARED((_NPAD, 128), jnp.float32),
    ],
)
def _deg_kernel(dst_hbm, ones_hbm, zer_hbm, out_hbm, idx_v, ones_v, acc):
    cid = lax.axis_index("c")
    sid = lax.axis_index("s")

    rows = pl.ds(sid * _RPT, _RPT)
    pltpu.sync_copy(zer_hbm, acc.at[rows])
    pltpu.sync_copy(ones_hbm, ones_v)
    plsc.subcore_barrier()

    def sup(si, _):
        pltpu.sync_copy(dst_hbm.at[cid, sid, pl.ds(si * _SS, _SS)], idx_v)
        for g in range(_SS):
            pltpu.sync_copy(ones_v, acc.at[idx_v.at[g]], add=True)
        return 0

    lax.fori_loop(0, _NSUP_D, sup, 0)
    plsc.subcore_barrier()
    pltpu.sync_copy(acc.at[rows], out_hbm.at[cid, rows])


@functools.partial(
    pl.kernel,
    out_type=[
        jax.ShapeDtypeStruct((_NPAD, 128), jnp.float32),
        jax.ShapeDtypeStruct((_NPAD, 128), jnp.float32),
    ],
    mesh=_sc_mesh,
    scratch_types=[
        pltpu.VMEM((_SS, _CH), jnp.int32),
        pltpu.VMEM((_SS, _CH), jnp.int32),
        pltpu.VMEM((_CH, 128), jnp.float32),
        pltpu.VMEM_SHARED((_NPAD, 128), jnp.float32),
        pltpu.SemaphoreType.DMA,
    ],
)
def _agg_kernel(srcp, dstp, zs_lo, zs_hi, zer_hbm, out_lo, out_hi,
                src_v, dst_v, buf_v, acc, sem):
    cid = lax.axis_index("c")
    sid = lax.axis_index("s")

    pltpu.sync_copy(zer_hbm, acc.at[pl.ds(sid * _RPT, _RPT)])
    plsc.subcore_barrier()

    def run_half(zs_hbm):
        def sup(si, _):
            pltpu.sync_copy(srcp.at[sid, pl.ds(si * _SS, _SS)], src_v)
            pltpu.sync_copy(dstp.at[sid, pl.ds(si * _SS, _SS)], dst_v)
            for g in range(_SS):
                pltpu.async_copy(zs_hbm.at[src_v.at[g]], buf_v, sem).wait()
                pltpu.sync_copy(buf_v, acc.at[dst_v.at[g]], add=True)
            return 0

        lax.fori_loop(0, _NSUP, sup, 0)

    @pl.when(cid == 0)
    def _():
        run_half(zs_lo)

    @pl.when(cid == 1)
    def _():
        run_half(zs_hi)

    plsc.subcore_barrier()
    rows = pl.ds(sid * _RPT, _RPT)

    @pl.when(cid == 0)
    def _():
        pltpu.sync_copy(acc.at[rows], out_lo.at[rows])

    @pl.when(cid == 1)
    def _():
        pltpu.sync_copy(acc.at[rows], out_hi.at[rows])


def _dis_of(d0, d1):
    return lax.rsqrt(d0[:, 0:1] + d1[:, 0:1] + 1.0)


def _zs_body(x_ref, d0_ref, d1_ref, a_ref, b_ref, w1_ref, zlo_ref, zhi_ref):
    dis = _dis_of(d0_ref, d1_ref)
    h0 = x_ref[...] * a_ref[...] + b_ref[...]
    z = jnp.dot(h0, w1_ref[...], preferred_element_type=jnp.float32) * dis
    zlo_ref[...] = z[:, :128]
    zhi_ref[...] = z[:, 128:]


def _post_conv(alo, ahi, zlo, zhi, dis, b, g, c, lg, lb):
    agg = jnp.concatenate([alo[...], ahi[...]], axis=1)
    zs = jnp.concatenate([zlo[...], zhi[...]], axis=1)
    conv = dis * (agg + zs) + b[...]
    t = jnp.maximum(conv, 0.0) * g[...] + c[...]
    m = jnp.mean(t, axis=1, keepdims=True)
    ct = t - m
    v = jnp.mean(ct * ct, axis=1, keepdims=True)
    return ct * lax.rsqrt(v + _EPS) * lg[...] + lb[...]


def _mid_body(alo, ahi, zlo, zhi, d0, d1, b1, g1, c1, lg1, lb1, w2_ref,
              h1_ref, z2lo_ref, z2hi_ref):
    dis = _dis_of(d0, d1)
    h1 = _post_conv(alo, ahi, zlo, zhi, dis, b1, g1, c1, lg1, lb1)
    h1_ref[...] = h1
    z2 = jnp.dot(h1, w2_ref[...], preferred_element_type=jnp.float32) * dis
    z2lo_ref[...] = z2[:, :128]
    z2hi_ref[...] = z2[:, 128:]


def _fin_body(alo, ahi, zlo, zhi, d0, d1, h1_ref, b2, g2, c2, lg2, lb2,
              wot_ref, bo_ref, out_ref):
    dis = _dis_of(d0, d1)
    h2 = _post_conv(alo, ahi, zlo, zhi, dis, b2, g2, c2, lg2, lb2) + h1_ref[...]
    wot = wot_ref[...]
    o0 = jnp.sum(h2 * wot[0:1, :], axis=1, keepdims=True)
    o1 = jnp.sum(h2 * wot[1:2, :], axis=1, keepdims=True)
    out_ref[...] = jnp.concatenate([o0, o1], axis=1) + bo_ref[...]


def _row_spec(cols):
    return pl.BlockSpec((_R, cols), lambda i: (i, 0))


def _full_spec(r, c):
    return pl.BlockSpec((r, c), lambda i: (0, 0))


_zs_call = pl.pallas_call(
    _zs_body,
    grid=(_GRID,),
    in_specs=[_row_spec(_D_IN), _row_spec(128), _row_spec(128),
              _full_spec(1, _D_IN), _full_spec(1, _D_IN),
              _full_spec(_D_IN, _D_H)],
    out_specs=[_row_spec(128), _row_spec(128)],
    out_shape=[jax.ShapeDtypeStruct((_N, 128), jnp.float32)] * 2,
)

_mid_call = pl.pallas_call(
    _mid_body,
    grid=(_GRID,),
    in_specs=[_row_spec(128), _row_spec(128), _row_spec(128), _row_spec(128),
              _row_spec(128), _row_spec(128)]
             + [_full_spec(1, _D_H)] * 5 + [_full_spec(_D_H, _D_H)],
    out_specs=[_row_spec(_D_H), _row_spec(128), _row_spec(128)],
    out_shape=[jax.ShapeDtypeStruct((_N, _D_H), jnp.float32),
               jax.ShapeDtypeStruct((_N, 128), jnp.float32),
               jax.ShapeDtypeStruct((_N, 128), jnp.float32)],
)

_fin_call = pl.pallas_call(
    _fin_body,
    grid=(_GRID,),
    in_specs=[_row_spec(128), _row_spec(128), _row_spec(128), _row_spec(128),
              _row_spec(128), _row_spec(128), _row_spec(_D_H)]
             + [_full_spec(1, _D_H)] * 5
             + [_full_spec(2, _D_H), _full_spec(1, 2)],
    out_specs=[_row_spec(2)],
    out_shape=[jax.ShapeDtypeStruct((_N, 2), jnp.float32)],
)


def kernel(x, edge_index, feature_scale, feature_bias, bn0_g, bn0_b,
           W1, b1, bn1_g, bn1_b, ln1_g, ln1_b,
           W2, b2, bn2_g, bn2_b, ln2_g, ln2_b, Wo, bo):
    s = np.float32(np.sqrt(1.0 + _EPS))
    src = edge_index[0]
    dst = edge_index[1]

    pad_a = _EPT_PAD - _EPT
    pada = jnp.broadcast_to(
        (_N + jnp.arange(_NS, dtype=jnp.int32))[:, None], (_NS, pad_a))
    srcp = jnp.concatenate(
        [src.reshape(_NS, _EPT), jnp.zeros((_NS, pad_a), jnp.int32)],
        axis=1).reshape(_NS, _NCH_A, _CH)
    dstp = jnp.concatenate(
        [dst.reshape(_NS, _EPT), pada], axis=1).reshape(_NS, _NCH_A, _CH)

    pad_d = _EPW_PAD - _EPW
    padd = jnp.broadcast_to(
        (_N + jnp.arange(_NC * _NS, dtype=jnp.int32)).reshape(_NC, _NS, 1),
        (_NC, _NS, pad_d))
    dstd = jnp.concatenate(
        [dst.reshape(_NC, _NS, _EPW), padd],
        axis=2).reshape(_NC, _NS, _NCH_D, _CH)

    ones128 = jnp.ones((_CH, 128), jnp.float32)
    zer128 = jnp.zeros((_RPT, 128), jnp.float32)

    degp = _deg_kernel(dstd, ones128, zer128)
    d0 = degp[0, :_N]
    d1 = degp[1, :_N]

    a0 = (feature_scale * bn0_g / s).reshape(1, _D_IN)
    b0 = (feature_bias * bn0_g / s + bn0_b).reshape(1, _D_IN)
    zlo, zhi = _zs_call(x, d0, d1, a0, b0, W1)

    a1lo, a1hi = _agg_kernel(srcp, dstp, zlo, zhi, zer128)

    g1 = (bn1_g / s).reshape(1, _D_H)
    h1, z2lo, z2hi = _mid_call(
        a1lo[:_N], a1hi[:_N], zlo, zhi, d0, d1,
        b1.reshape(1, _D_H), g1, bn1_b.reshape(1, _D_H),
        ln1_g.reshape(1, _D_H), ln1_b.reshape(1, _D_H), W2)

    a2lo, a2hi = _agg_kernel(srcp, dstp, z2lo, z2hi, zer128)

    g2 = (bn2_g / s).reshape(1, _D_H)
    (out,) = _fin_call(
        a2lo[:_N], a2hi[:_N], z2lo, z2hi, d0, d1, h1,
        b2.reshape(1, _D_H), g2, bn2_b.reshape(1, _D_H),
        ln2_g.reshape(1, _D_H), ln2_b.reshape(1, _D_H),
        Wo.T, bo.reshape(1, 2))
    return out


# speedup vs baseline: 9.2339x; 1.1496x over previous
import functools

import jax
import jax.numpy as jnp
import numpy as np
from jax import lax
from jax.experimental import pallas as pl
from jax.experimental.pallas import tpu as pltpu
from jax.experimental.pallas import tpu_sc as plsc

_N = 10000
_E = 320000
_D_IN = 128
_D_H = 256
_EPS = 1e-5

_NC = 2
_NS = 16
_NPAD = 10240
_RPT = _NPAD // _NS
_CH = 128

_EPT = _E // _NS
_SS = 8
_NCH_A = 160
_NSUP = _NCH_A // _SS
_EPT_PAD = _NCH_A * _CH

_EPW = _E // (_NC * _NS)
_NCH_D = 80
_NSUP_D = _NCH_D // _SS
_EPW_PAD = _NCH_D * _CH

_R = 1000
_GRID = _N // _R

_sc_mesh = plsc.VectorSubcoreMesh(core_axis_name="c", subcore_axis_name="s")


@functools.partial(
    pl.kernel,
    out_type=jax.ShapeDtypeStruct((_NC, _NPAD, 128), jnp.float32),
    mesh=_sc_mesh,
    scratch_types=[
        pltpu.VMEM((_SS, _CH), jnp.int32),
        pltpu.VMEM((_CH, 128), jnp.float32),
        pltpu.VMEM_SHARED((_NPAD, 128), jnp.float32),
        pltpu.SemaphoreType.DMA,
    ],
)
def _deg_kernel(dst_hbm, ones_hbm, zer_hbm, out_hbm, idx_v, ones_v, acc, ssem):
    cid = lax.axis_index("c")
    sid = lax.axis_index("s")

    rows = pl.ds(sid * _RPT, _RPT)
    pltpu.sync_copy(zer_hbm, acc.at[rows])
    pltpu.sync_copy(ones_hbm, ones_v)
    plsc.subcore_barrier()

    def sup(si, _):
        pltpu.sync_copy(dst_hbm.at[cid, sid, pl.ds(si * _SS, _SS)], idx_v)
        descs = [pltpu.async_copy(ones_v, acc.at[idx_v.at[g]], ssem, add=True)
                 for g in range(_SS)]
        for d in descs:
            d.wait()
        return 0

    lax.fori_loop(0, _NSUP_D, sup, 0)
    plsc.subcore_barrier()
    pltpu.sync_copy(acc.at[rows], out_hbm.at[cid, rows])


@functools.partial(
    pl.kernel,
    out_type=[
        jax.ShapeDtypeStruct((_NPAD, 128), jnp.float32),
        jax.ShapeDtypeStruct((_NPAD, 128), jnp.float32),
    ],
    mesh=_sc_mesh,
    scratch_types=[
        pltpu.VMEM((_SS, _CH), jnp.int32),
        pltpu.VMEM((_SS, _CH), jnp.int32),
        pltpu.VMEM((_CH, 128), jnp.float32),
        pltpu.VMEM((_CH, 128), jnp.float32),
        pltpu.VMEM_SHARED((_NPAD, 128), jnp.float32),
        pltpu.SemaphoreType.DMA,
        pltpu.SemaphoreType.DMA,
    ],
)
def _agg_kernel(srcp, dstp, zs_lo, zs_hi, zer_hbm, out_lo, out_hi,
                src_v, dst_v, buf0, buf1, acc, sem0, sem1):
    cid = lax.axis_index("c")
    sid = lax.axis_index("s")

    pltpu.sync_copy(zer_hbm, acc.at[pl.ds(sid * _RPT, _RPT)])
    plsc.subcore_barrier()

    bufs = [buf0, buf1]
    sems = [sem0, sem1]

    def run_half(zs_hbm):
        def sup(si, _):
            pltpu.sync_copy(srcp.at[sid, pl.ds(si * _SS, _SS)], src_v)
            pltpu.sync_copy(dstp.at[sid, pl.ds(si * _SS, _SS)], dst_v)
            descs = [None] * _SS
            descs[0] = pltpu.async_copy(
                zs_hbm.at[src_v.at[0]], bufs[0], sems[0])
            for g in range(_SS):
                if g + 1 < _SS:
                    descs[g + 1] = pltpu.async_copy(
                        zs_hbm.at[src_v.at[g + 1]],
                        bufs[(g + 1) % 2], sems[(g + 1) % 2])
                descs[g].wait()
                pltpu.sync_copy(bufs[g % 2], acc.at[dst_v.at[g]], add=True)
            return 0

        lax.fori_loop(0, _NSUP, sup, 0)

    @pl.when(cid == 0)
    def _():
        run_half(zs_lo)

    @pl.when(cid == 1)
    def _():
        run_half(zs_hi)

    plsc.subcore_barrier()
    rows = pl.ds(sid * _RPT, _RPT)

    @pl.when(cid == 0)
    def _():
        pltpu.sync_copy(acc.at[rows], out_lo.at[rows])

    @pl.when(cid == 1)
    def _():
        pltpu.sync_copy(acc.at[rows], out_hi.at[rows])


def _dis_of(d0, d1):
    return lax.rsqrt(d0[:, 0:1] + d1[:, 0:1] + 1.0)


def _zs_body(x_ref, d0_ref, d1_ref, a_ref, b_ref, w1_ref, zlo_ref, zhi_ref):
    dis = _dis_of(d0_ref, d1_ref)
    h0 = x_ref[...] * a_ref[...] + b_ref[...]
    z = jnp.dot(h0, w1_ref[...], preferred_element_type=jnp.float32) * dis
    zlo_ref[...] = z[:, :128]
    zhi_ref[...] = z[:, 128:]


def _post_conv(alo, ahi, zlo, zhi, dis, b, g, c, lg, lb):
    agg = jnp.concatenate([alo[...], ahi[...]], axis=1)
    zs = jnp.concatenate([zlo[...], zhi[...]], axis=1)
    conv = dis * (agg + zs) + b[...]
    t = jnp.maximum(conv, 0.0) * g[...] + c[...]
    m = jnp.mean(t, axis=1, keepdims=True)
    ct = t - m
    v = jnp.mean(ct * ct, axis=1, keepdims=True)
    return ct * lax.rsqrt(v + _EPS) * lg[...] + lb[...]


def _mid_body(alo, ahi, zlo, zhi, d0, d1, b1, g1, c1, lg1, lb1, w2_ref,
              h1_ref, z2lo_ref, z2hi_ref):
    dis = _dis_of(d0, d1)
    h1 = _post_conv(alo, ahi, zlo, zhi, dis, b1, g1, c1, lg1, lb1)
    h1_ref[...] = h1
    z2 = jnp.dot(h1, w2_ref[...], preferred_element_type=jnp.float32) * dis
    z2lo_ref[...] = z2[:, :128]
    z2hi_ref[...] = z2[:, 128:]


def _fin_body(alo, ahi, zlo, zhi, d0, d1, h1_ref, b2, g2, c2, lg2, lb2,
              wot_ref, bo_ref, out_ref):
    dis = _dis_of(d0, d1)
    h2 = _post_conv(alo, ahi, zlo, zhi, dis, b2, g2, c2, lg2, lb2) + h1_ref[...]
    wot = wot_ref[...]
    o0 = jnp.sum(h2 * wot[0:1, :], axis=1, keepdims=True)
    o1 = jnp.sum(h2 * wot[1:2, :], axis=1, keepdims=True)
    out_ref[...] = jnp.concatenate([o0, o1], axis=1) + bo_ref[...]


def _row_spec(cols):
    return pl.BlockSpec((_R, cols), lambda i: (i, 0))


def _full_spec(r, c):
    return pl.BlockSpec((r, c), lambda i: (0, 0))


_zs_call = pl.pallas_call(
    _zs_body,
    grid=(_GRID,),
    in_specs=[_row_spec(_D_IN), _row_spec(128), _row_spec(128),
              _full_spec(1, _D_IN), _full_spec(1, _D_IN),
              _full_spec(_D_IN, _D_H)],
    out_specs=[_row_spec(128), _row_spec(128)],
    out_shape=[jax.ShapeDtypeStruct((_N, 128), jnp.float32)] * 2,
)

_mid_call = pl.pallas_call(
    _mid_body,
    grid=(_GRID,),
    in_specs=[_row_spec(128), _row_spec(128), _row_spec(128), _row_spec(128),
              _row_spec(128), _row_spec(128)]
             + [_full_spec(1, _D_H)] * 5 + [_full_spec(_D_H, _D_H)],
    out_specs=[_row_spec(_D_H), _row_spec(128), _row_spec(128)],
    out_shape=[jax.ShapeDtypeStruct((_N, _D_H), jnp.float32),
               jax.ShapeDtypeStruct((_N, 128), jnp.float32),
               jax.ShapeDtypeStruct((_N, 128), jnp.float32)],
)

_fin_call = pl.pallas_call(
    _fin_body,
    grid=(_GRID,),
    in_specs=[_row_spec(128), _row_spec(128), _row_spec(128), _row_spec(128),
              _row_spec(128), _row_spec(128), _row_spec(_D_H)]
             + [_full_spec(1, _D_H)] * 5
             + [_full_spec(2, _D_H), _full_spec(1, 2)],
    out_specs=[_row_spec(2)],
    out_shape=[jax.ShapeDtypeStruct((_N, 2), jnp.float32)],
)


def kernel(x, edge_index, feature_scale, feature_bias, bn0_g, bn0_b,
           W1, b1, bn1_g, bn1_b, ln1_g, ln1_b,
           W2, b2, bn2_g, bn2_b, ln2_g, ln2_b, Wo, bo):
    s = np.float32(np.sqrt(1.0 + _EPS))
    src = edge_index[0]
    dst = edge_index[1]

    pad_a = _EPT_PAD - _EPT
    pada = jnp.broadcast_to(
        (_N + jnp.arange(_NS, dtype=jnp.int32))[:, None], (_NS, pad_a))
    srcp = jnp.concatenate(
        [src.reshape(_NS, _EPT), jnp.zeros((_NS, pad_a), jnp.int32)],
        axis=1).reshape(_NS, _NCH_A, _CH)
    dstp = jnp.concatenate(
        [dst.reshape(_NS, _EPT), pada], axis=1).reshape(_NS, _NCH_A, _CH)

    pad_d = _EPW_PAD - _EPW
    padd = jnp.broadcast_to(
        (_N + jnp.arange(_NC * _NS, dtype=jnp.int32)).reshape(_NC, _NS, 1),
        (_NC, _NS, pad_d))
    dstd = jnp.concatenate(
        [dst.reshape(_NC, _NS, _EPW), padd],
        axis=2).reshape(_NC, _NS, _NCH_D, _CH)

    ones128 = jnp.ones((_CH, 128), jnp.float32)
    zer128 = jnp.zeros((_RPT, 128), jnp.float32)

    degp = _deg_kernel(dstd, ones128, zer128)
    d0 = degp[0, :_N]
    d1 = degp[1, :_N]

    a0 = (feature_scale * bn0_g / s).reshape(1, _D_IN)
    b0 = (feature_bias * bn0_g / s + bn0_b).reshape(1, _D_IN)
    zlo, zhi = _zs_call(x, d0, d1, a0, b0, W1)

    a1lo, a1hi = _agg_kernel(srcp, dstp, zlo, zhi, zer128)

    g1 = (bn1_g / s).reshape(1, _D_H)
    h1, z2lo, z2hi = _mid_call(
        a1lo[:_N], a1hi[:_N], zlo, zhi, d0, d1,
        b1.reshape(1, _D_H), g1, bn1_b.reshape(1, _D_H),
        ln1_g.reshape(1, _D_H), ln1_b.reshape(1, _D_H), W2)

    a2lo, a2hi = _agg_kernel(srcp, dstp, z2lo, z2hi, zer128)

    g2 = (bn2_g / s).reshape(1, _D_H)
    (out,) = _fin_call(
        a2lo[:_N], a2hi[:_N], z2lo, z2hi, d0, d1, h1,
        b2.reshape(1, _D_H), g2, bn2_b.reshape(1, _D_H),
        ln2_g.reshape(1, _D_H), ln2_b.reshape(1, _D_H),
        Wo.T, bo.reshape(1, 2))
    return out


# speedup vs baseline: 9.2364x; 1.0003x over previous
import functools

import jax
import jax.numpy as jnp
import numpy as np
from jax import lax
from jax.experimental import pallas as pl
from jax.experimental.pallas import tpu as pltpu
from jax.experimental.pallas import tpu_sc as plsc

_N = 10000
_E = 320000
_D_IN = 128
_D_H = 256
_EPS = 1e-5

_NC = 2
_NS = 16
_NPAD = 10240
_RPT = _NPAD // _NS
_CH = 128

_EPT = _E // _NS
_SS = 8
_NCH_A = 160
_NSUP = _NCH_A // _SS
_EPT_PAD = _NCH_A * _CH

_EPW = _E // (_NC * _NS)
_NCH_D = 80
_NSUP_D = _NCH_D // _SS
_EPW_PAD = _NCH_D * _CH

_R = 1000
_GRID = _N // _R

_sc_mesh = plsc.VectorSubcoreMesh(core_axis_name="c", subcore_axis_name="s")


@functools.partial(
    pl.kernel,
    out_type=jax.ShapeDtypeStruct((_NC, _NPAD, 128), jnp.float32),
    mesh=_sc_mesh,
    scratch_types=[
        pltpu.VMEM((_SS, _CH), jnp.int32),
        pltpu.VMEM((_CH, 128), jnp.float32),
        pltpu.VMEM_SHARED((_NPAD, 128), jnp.float32),
        pltpu.SemaphoreType.DMA,
    ],
)
def _deg_kernel(dst_hbm, ones_hbm, zer_hbm, out_hbm, idx_v, ones_v, acc, ssem):
    cid = lax.axis_index("c")
    sid = lax.axis_index("s")

    rows = pl.ds(sid * _RPT, _RPT)
    pltpu.sync_copy(zer_hbm, acc.at[rows])
    pltpu.sync_copy(ones_hbm, ones_v)
    plsc.subcore_barrier()

    def sup(si, _):
        pltpu.sync_copy(dst_hbm.at[cid, sid, pl.ds(si * _SS, _SS)], idx_v)
        descs = [pltpu.async_copy(ones_v, acc.at[idx_v.at[g]], ssem, add=True)
                 for g in range(_SS)]
        for d in descs:
            d.wait()
        return 0

    lax.fori_loop(0, _NSUP_D, sup, 0)
    plsc.subcore_barrier()
    pltpu.sync_copy(acc.at[rows], out_hbm.at[cid, rows])


@functools.partial(
    pl.kernel,
    out_type=[
        jax.ShapeDtypeStruct((_NPAD, 128), jnp.float32),
        jax.ShapeDtypeStruct((_NPAD, 128), jnp.float32),
    ],
    mesh=_sc_mesh,
    scratch_types=[
        pltpu.VMEM((_SS, _CH), jnp.int32),
        pltpu.VMEM((_SS, _CH), jnp.int32),
        pltpu.VMEM((_CH, 128), jnp.float32),
        pltpu.VMEM((_CH, 128), jnp.float32),
        pltpu.VMEM_SHARED((_NPAD, 128), jnp.float32),
        pltpu.SemaphoreType.DMA,
        pltpu.SemaphoreType.DMA,
        pltpu.SemaphoreType.DMA,
        pltpu.SemaphoreType.DMA,
    ],
)
def _agg_kernel(srcp, dstp, zs_lo, zs_hi, zer_hbm, out_lo, out_hi,
                src_v, dst_v, buf0, buf1, acc, gsem0, gsem1, ssem0, ssem1):
    cid = lax.axis_index("c")
    sid = lax.axis_index("s")

    pltpu.sync_copy(zer_hbm, acc.at[pl.ds(sid * _RPT, _RPT)])
    plsc.subcore_barrier()

    bufs = [buf0, buf1]
    gsems = [gsem0, gsem1]
    ssems = [ssem0, ssem1]

    def run_half(zs_hbm):
        def sup(si, _):
            pltpu.sync_copy(srcp.at[sid, pl.ds(si * _SS, _SS)], src_v)
            pltpu.sync_copy(dstp.at[sid, pl.ds(si * _SS, _SS)], dst_v)
            gd = [None] * _SS
            sd = [None] * _SS
            for g in range(2):
                gd[g] = pltpu.async_copy(
                    zs_hbm.at[src_v.at[g]], bufs[g % 2], gsems[g % 2])
            for g in range(_SS):
                gd[g].wait()
                sd[g] = pltpu.async_copy(
                    bufs[g % 2], acc.at[dst_v.at[g]], ssems[g % 2], add=True)
                if g + 2 < _SS:
                    sd[g].wait()
                    gd[g + 2] = pltpu.async_copy(
                        zs_hbm.at[src_v.at[g + 2]], bufs[g % 2], gsems[g % 2])
            for g in range(_SS - 2, _SS):
                sd[g].wait()
            return 0

        lax.fori_loop(0, _NSUP, sup, 0)

    @pl.when(cid == 0)
    def _():
        run_half(zs_lo)

    @pl.when(cid == 1)
    def _():
        run_half(zs_hi)

    plsc.subcore_barrier()
    rows = pl.ds(sid * _RPT, _RPT)

    @pl.when(cid == 0)
    def _():
        pltpu.sync_copy(acc.at[rows], out_lo.at[rows])

    @pl.when(cid == 1)
    def _():
        pltpu.sync_copy(acc.at[rows], out_hi.at[rows])


def _dis_of(d0, d1):
    return lax.rsqrt(d0[:, 0:1] + d1[:, 0:1] + 1.0)


def _zs_body(x_ref, d0_ref, d1_ref, a_ref, b_ref, w1_ref, zlo_ref, zhi_ref):
    dis = _dis_of(d0_ref, d1_ref)
    h0 = x_ref[...] * a_ref[...] + b_ref[...]
    z = jnp.dot(h0, w1_ref[...], preferred_element_type=jnp.float32) * dis
    zlo_ref[...] = z[:, :128]
    zhi_ref[...] = z[:, 128:]


def _post_conv(alo, ahi, zlo, zhi, dis, b, g, c, lg, lb):
    agg = jnp.concatenate([alo[...], ahi[...]], axis=1)
    zs = jnp.concatenate([zlo[...], zhi[...]], axis=1)
    conv = dis * (agg + zs) + b[...]
    t = jnp.maximum(conv, 0.0) * g[...] + c[...]
    m = jnp.mean(t, axis=1, keepdims=True)
    ct = t - m
    v = jnp.mean(ct * ct, axis=1, keepdims=True)
    return ct * lax.rsqrt(v + _EPS) * lg[...] + lb[...]


def _mid_body(alo, ahi, zlo, zhi, d0, d1, b1, g1, c1, lg1, lb1, w2_ref,
              h1_ref, z2lo_ref, z2hi_ref):
    dis = _dis_of(d0, d1)
    h1 = _post_conv(alo, ahi, zlo, zhi, dis, b1, g1, c1, lg1, lb1)
    h1_ref[...] = h1
    z2 = jnp.dot(h1, w2_ref[...], preferred_element_type=jnp.float32) * dis
    z2lo_ref[...] = z2[:, :128]
    z2hi_ref[...] = z2[:, 128:]


def _fin_body(alo, ahi, zlo, zhi, d0, d1, h1_ref, b2, g2, c2, lg2, lb2,
              wot_ref, bo_ref, out_ref):
    dis = _dis_of(d0, d1)
    h2 = _post_conv(alo, ahi, zlo, zhi, dis, b2, g2, c2, lg2, lb2) + h1_ref[...]
    wot = wot_ref[...]
    o0 = jnp.sum(h2 * wot[0:1, :], axis=1, keepdims=True)
    o1 = jnp.sum(h2 * wot[1:2, :], axis=1, keepdims=True)
    out_ref[...] = jnp.concatenate([o0, o1], axis=1) + bo_ref[...]


def _row_spec(cols):
    return pl.BlockSpec((_R, cols), lambda i: (i, 0))


def _full_spec(r, c):
    return pl.BlockSpec((r, c), lambda i: (0, 0))


_zs_call = pl.pallas_call(
    _zs_body,
    grid=(_GRID,),
    in_specs=[_row_spec(_D_IN), _row_spec(128), _row_spec(128),
              _full_spec(1, _D_IN), _full_spec(1, _D_IN),
              _full_spec(_D_IN, _D_H)],
    out_specs=[_row_spec(128), _row_spec(128)],
    out_shape=[jax.ShapeDtypeStruct((_N, 128), jnp.float32)] * 2,
)

_mid_call = pl.pallas_call(
    _mid_body,
    grid=(_GRID,),
    in_specs=[_row_spec(128), _row_spec(128), _row_spec(128), _row_spec(128),
              _row_spec(128), _row_spec(128)]
             + [_full_spec(1, _D_H)] * 5 + [_full_spec(_D_H, _D_H)],
    out_specs=[_row_spec(_D_H), _row_spec(128), _row_spec(128)],
    out_shape=[jax.ShapeDtypeStruct((_N, _D_H), jnp.float32),
               jax.ShapeDtypeStruct((_N, 128), jnp.float32),
               jax.ShapeDtypeStruct((_N, 128), jnp.float32)],
)

_fin_call = pl.pallas_call(
    _fin_body,
    grid=(_GRID,),
    in_specs=[_row_spec(128), _row_spec(128), _row_spec(128), _row_spec(128),
              _row_spec(128), _row_spec(128), _row_spec(_D_H)]
             + [_full_spec(1, _D_H)] * 5
             + [_full_spec(2, _D_H), _full_spec(1, 2)],
    out_specs=[_row_spec(2)],
    out_shape=[jax.ShapeDtypeStruct((_N, 2), jnp.float32)],
)


def kernel(x, edge_index, feature_scale, feature_bias, bn0_g, bn0_b,
           W1, b1, bn1_g, bn1_b, ln1_g, ln1_b,
           W2, b2, bn2_g, bn2_b, ln2_g, ln2_b, Wo, bo):
    s = np.float32(np.sqrt(1.0 + _EPS))
    src = edge_index[0]
    dst = edge_index[1]

    pad_a = _EPT_PAD - _EPT
    pada = jnp.broadcast_to(
        (_N + jnp.arange(_NS, dtype=jnp.int32))[:, None], (_NS, pad_a))
    srcp = jnp.concatenate(
        [src.reshape(_NS, _EPT), jnp.zeros((_NS, pad_a), jnp.int32)],
        axis=1).reshape(_NS, _NCH_A, _CH)
    dstp = jnp.concatenate(
        [dst.reshape(_NS, _EPT), pada], axis=1).reshape(_NS, _NCH_A, _CH)

    pad_d = _EPW_PAD - _EPW
    padd = jnp.broadcast_to(
        (_N + jnp.arange(_NC * _NS, dtype=jnp.int32)).reshape(_NC, _NS, 1),
        (_NC, _NS, pad_d))
    dstd = jnp.concatenate(
        [dst.reshape(_NC, _NS, _EPW), padd],
        axis=2).reshape(_NC, _NS, _NCH_D, _CH)

    ones128 = jnp.ones((_CH, 128), jnp.float32)
    zer128 = jnp.zeros((_RPT, 128), jnp.float32)

    degp = _deg_kernel(dstd, ones128, zer128)
    d0 = degp[0, :_N]
    d1 = degp[1, :_N]

    a0 = (feature_scale * bn0_g / s).reshape(1, _D_IN)
    b0 = (feature_bias * bn0_g / s + bn0_b).reshape(1, _D_IN)
    zlo, zhi = _zs_call(x, d0, d1, a0, b0, W1)

    a1lo, a1hi = _agg_kernel(srcp, dstp, zlo, zhi, zer128)

    g1 = (bn1_g / s).reshape(1, _D_H)
    h1, z2lo, z2hi = _mid_call(
        a1lo[:_N], a1hi[:_N], zlo, zhi, d0, d1,
        b1.reshape(1, _D_H), g1, bn1_b.reshape(1, _D_H),
        ln1_g.reshape(1, _D_H), ln1_b.reshape(1, _D_H), W2)

    a2lo, a2hi = _agg_kernel(srcp, dstp, z2lo, z2hi, zer128)

    g2 = (bn2_g / s).reshape(1, _D_H)
    (out,) = _fin_call(
        a2lo[:_N], a2hi[:_N], z2lo, z2hi, d0, d1, h1,
        b2.reshape(1, _D_H), g2, bn2_b.reshape(1, _D_H),
        ln2_g.reshape(1, _D_H), ln2_b.reshape(1, _D_H),
        Wo.T, bo.reshape(1, 2))
    return out
